# baseline (speedup 1.0000x reference)
import functools

import jax
import jax.numpy as jnp
from jax import lax
from jax.experimental import pallas as pl
from jax.experimental.pallas import tpu as pltpu
from jax.experimental.pallas import tpu_sc as plsc

NC = 2
NS = 16
L = 16
NWPAD = 576


def _neumf_sc(B, MF, E, L0, L1):
    NW = NC * NS
    bpw = B // NW
    nblk = bpw // L
    nw1 = L1 * L0
    mesh = plsc.VectorSubcoreMesh(
        core_axis_name="c", subcore_axis_name="s",
        num_cores=NC, num_subcores=NS)

    stg_t = pltpu.VMEM((4, L, 128), jnp.float32)

    @functools.partial(
        pl.kernel,
        mesh=mesh,
        compiler_params=pltpu.CompilerParams(
            needs_layout_passes=False, use_tc_tiling_on_sc=True),
        out_type=jax.ShapeDtypeStruct((B,), jnp.float32),
        scratch_types=[
            pltpu.VMEM((bpw,), jnp.int32),
            pltpu.VMEM((bpw,), jnp.int32),
            pltpu.VMEM_SHARED((NS, bpw), jnp.int32),
            pltpu.VMEM_SHARED((NS, bpw), jnp.int32),
            pltpu.SMEM((bpw,), jnp.int32),
            pltpu.SMEM((bpw,), jnp.int32),
            pltpu.VMEM((NWPAD,), jnp.float32),
            pltpu.VMEM_SHARED((NS, NWPAD), jnp.float32),
            pltpu.SMEM((NWPAD,), jnp.float32),
            stg_t,
            stg_t,
            stg_t,
            stg_t,
            pltpu.VMEM((bpw,), jnp.float32),
            pltpu.SemaphoreType.DMA((2,)),
        ],
    )
    def neumf(uidx_hbm, iidx_hbm, ugT_hbm, igT_hbm, umT_hbm, imT_hbm, w_hbm,
              out_hbm,
              uidx_v, iidx_v, uidx_sh, iidx_sh, uidx_s, iidx_s,
              w_v, w_sh, w_s, sug, sig, sum_, sim, out_v, sems):
        sid = lax.axis_index("s")
        wid = sid * NC + lax.axis_index("c")
        base = wid * bpw

        pltpu.sync_copy(uidx_hbm.at[pl.ds(base, bpw)], uidx_v)
        pltpu.sync_copy(iidx_hbm.at[pl.ds(base, bpw)], iidx_v)
        pltpu.sync_copy(w_hbm, w_v)
        pltpu.sync_copy(uidx_v, uidx_sh.at[sid])
        pltpu.sync_copy(iidx_v, iidx_sh.at[sid])
        pltpu.sync_copy(w_v, w_sh.at[sid])
        pltpu.sync_copy(uidx_sh.at[sid], uidx_s)
        pltpu.sync_copy(iidx_sh.at[sid], iidx_s)
        pltpu.sync_copy(w_sh.at[sid], w_s)

        NDMA = 8

        def fire(s, p):
            sem = sems.at[p]

            def one(i, _):
                ru = uidx_s[s * L + i]
                ri = iidx_s[s * L + i]
                ru0 = pl.multiple_of(ru & -16, 16)
                ri0 = pl.multiple_of(ri & -16, 16)
                q = p * 2 + (i >> 3)
                lb = (i & 7) * L
                for h in (0, 8):
                    pltpu.async_copy(
                        ugT_hbm.at[pl.ds(h, 8), pl.ds(ru0, L)],
                        sug.at[q, pl.ds(h, 8), pl.ds(lb, L)], sem)
                    pltpu.async_copy(
                        igT_hbm.at[pl.ds(h, 8), pl.ds(ri0, L)],
                        sig.at[q, pl.ds(h, 8), pl.ds(lb, L)], sem)
                    pltpu.async_copy(
                        umT_hbm.at[pl.ds(h, 8), pl.ds(ru0, L)],
                        sum_.at[q, pl.ds(h, 8), pl.ds(lb, L)], sem)
                    pltpu.async_copy(
                        imT_hbm.at[pl.ds(h, 8), pl.ds(ri0, L)],
                        sim.at[q, pl.ds(h, 8), pl.ds(lb, L)], sem)
                return 0
            lax.fori_loop(0, L, one, 0)

        def drain(p):
            def w(i, _):
                pltpu.make_async_copy(
                    ugT_hbm.at[pl.ds(0, 8), pl.ds(0, L)],
                    sug.at[0, pl.ds(0, 8), pl.ds(0, L)],
                    sems.at[p]).wait()
                return 0
            lax.fori_loop(0, NDMA * L, w, 0)

        b1_splat = tuple(jnp.full((L,), w_s[nw1 + j], jnp.float32)
                         for j in range(L1))
        bout_splat = jnp.full((L,), w_s[nw1 + L1 + MF + L1], jnp.float32)
        lanes = lax.iota(jnp.int32, L)

        def compute(t, p):
            slotq = (lanes >> 3) + p * 2
            lb = (lanes & 7) * L
            rlo_u = lb + (uidx_v[pl.ds(t * L, L)] & 15)
            rlo_i = lb + (iidx_v[pl.ds(t * L, L)] & 15)
            acc = bout_splat
            for d in range(MF):
                dsp = jnp.full((L,), d, jnp.int32)
                ug_d = plsc.load_gather(sug, [slotq, dsp, rlo_u])
                ig_d = plsc.load_gather(sig, [slotq, dsp, rlo_i])
                acc = acc + (ug_d * ig_d) * w_s[nw1 + L1 + d]

            xs = []
            for k in range(E):
                xs.append(plsc.load_gather(
                    sum_, [slotq, jnp.full((L,), k, jnp.int32), rlo_u]))
            for k in range(L0 - E):
                xs.append(plsc.load_gather(
                    sim, [slotq, jnp.full((L,), k, jnp.int32), rlo_i]))
            hs = list(b1_splat)
            for k in range(L0):
                for j in range(L1):
                    hs[j] = hs[j] + xs[k] * w_s[j * L0 + k]
            for j in range(L1):
                acc = acc + jnp.maximum(hs[j], 0.0) * w_s[nw1 + L1 + MF + j]
            out_v[pl.ds(t * L, L)] = acc

        fire(0, 0)

        def body(t, _):
            p = lax.rem(t, 2)

            @pl.when(t + 1 < nblk)
            def _():
                fire(t + 1, 1 - p)

            drain(p)
            compute(t, p)
            return 0

        lax.fori_loop(0, nblk, body, 0)
        pltpu.sync_copy(out_v, out_hbm.at[pl.ds(base, bpw)])

    return neumf


def kernel(user_indices, item_indices, ue_gmf, ie_gmf, ue_mlp, ie_mlp,
           W1, b1, Wout, bout):
    B = user_indices.shape[0]
    MF = ue_gmf.shape[1]
    E = ue_mlp.shape[1]
    L1, L0 = W1.shape
    w_all = jnp.concatenate(
        [W1.reshape(-1), b1.reshape(-1), Wout.reshape(-1), bout.reshape(-1)])
    w_all = jnp.pad(w_all, (0, NWPAD - w_all.shape[0]))
    fn = _neumf_sc(B, MF, E, L0, L1)
    out = fn(user_indices.astype(jnp.int32), item_indices.astype(jnp.int32),
             ue_gmf.T, ie_gmf.T, ue_mlp.T, ie_mlp.T, w_all)
    return out.reshape(B, 1)


# speedup vs baseline: 1.5890x; 1.5890x over previous
import functools

import jax
import jax.numpy as jnp
from jax import lax
from jax.experimental import pallas as pl
from jax.experimental.pallas import tpu as pltpu
from jax.experimental.pallas import tpu_sc as plsc

NC = 2
NS = 16
L = 16
NWPAD = 576


def _neumf_sc(B, MF, E, L0, L1):
    NW = NC * NS
    bpw = B // NW
    nblk = bpw // L
    nw1 = L1 * L0
    mesh = plsc.VectorSubcoreMesh(
        core_axis_name="c", subcore_axis_name="s",
        num_cores=NC, num_subcores=NS)

    stg_t = pltpu.VMEM((4, L, 128), jnp.float32)

    @functools.partial(
        pl.kernel,
        mesh=mesh,
        compiler_params=pltpu.CompilerParams(
            needs_layout_passes=False, use_tc_tiling_on_sc=True),
        out_type=jax.ShapeDtypeStruct((B,), jnp.float32),
        scratch_types=[
            pltpu.VMEM((bpw,), jnp.int32),
            pltpu.VMEM((bpw,), jnp.int32),
            pltpu.VMEM_SHARED((NS, bpw), jnp.int32),
            pltpu.VMEM_SHARED((NS, bpw), jnp.int32),
            pltpu.SMEM((bpw,), jnp.int32),
            pltpu.SMEM((bpw,), jnp.int32),
            pltpu.VMEM((NWPAD,), jnp.float32),
            pltpu.VMEM_SHARED((NS, NWPAD), jnp.float32),
            pltpu.SMEM((NWPAD,), jnp.float32),
            stg_t,
            stg_t,
            stg_t,
            stg_t,
            pltpu.VMEM((bpw,), jnp.float32),
            pltpu.SemaphoreType.DMA((2,)),
        ],
    )
    def neumf(uidx_hbm, iidx_hbm, ugT_hbm, igT_hbm, umT_hbm, imT_hbm, w_hbm,
              out_hbm,
              uidx_v, iidx_v, uidx_sh, iidx_sh, uidx_s, iidx_s,
              w_v, w_sh, w_s, sug, sig, sum_, sim, out_v, sems):
        sid = lax.axis_index("s")
        wid = sid * NC + lax.axis_index("c")
        base = wid * bpw

        pltpu.sync_copy(uidx_hbm.at[pl.ds(base, bpw)], uidx_v)
        pltpu.sync_copy(iidx_hbm.at[pl.ds(base, bpw)], iidx_v)
        pltpu.sync_copy(w_hbm, w_v)
        pltpu.sync_copy(uidx_v, uidx_sh.at[sid])
        pltpu.sync_copy(iidx_v, iidx_sh.at[sid])
        pltpu.sync_copy(w_v, w_sh.at[sid])
        pltpu.sync_copy(uidx_sh.at[sid], uidx_s)
        pltpu.sync_copy(iidx_sh.at[sid], iidx_s)
        pltpu.sync_copy(w_sh.at[sid], w_s)

        NDMA = 8

        def fire(s, p):
            sem = sems.at[p]

            def one(i, _):
                ru = uidx_s[s * L + i]
                ri = iidx_s[s * L + i]
                ru0 = pl.multiple_of(ru & -16, 16)
                ri0 = pl.multiple_of(ri & -16, 16)
                q = p * 2 + (i >> 3)
                lb = (i & 7) * L
                for h in (0, 8):
                    pltpu.async_copy(
                        ugT_hbm.at[pl.ds(h, 8), pl.ds(ru0, L)],
                        sug.at[q, pl.ds(h, 8), pl.ds(lb, L)], sem)
                    pltpu.async_copy(
                        igT_hbm.at[pl.ds(h, 8), pl.ds(ri0, L)],
                        sig.at[q, pl.ds(h, 8), pl.ds(lb, L)], sem)
                    pltpu.async_copy(
                        umT_hbm.at[pl.ds(h, 8), pl.ds(ru0, L)],
                        sum_.at[q, pl.ds(h, 8), pl.ds(lb, L)], sem)
                    pltpu.async_copy(
                        imT_hbm.at[pl.ds(h, 8), pl.ds(ri0, L)],
                        sim.at[q, pl.ds(h, 8), pl.ds(lb, L)], sem)
                return 0
            lax.fori_loop(0, L, one, 0)

        def drain(p):
            def w(i, _):
                pltpu.make_async_copy(
                    ugT_hbm.at[pl.ds(0, 8), pl.ds(0, L)],
                    sug.at[0, pl.ds(0, 8), pl.ds(0, L)],
                    sems.at[p]).wait()
                return 0
            lax.fori_loop(0, NDMA * L, w, 0)

        b1_splat = tuple(jnp.full((L,), w_s[nw1 + j], jnp.float32)
                         for j in range(L1))
        bout_splat = jnp.full((L,), w_s[nw1 + L1 + MF + L1], jnp.float32)
        lanes = lax.iota(jnp.int32, L)

        def compute(t, p):
            slotq = (lanes >> 3) + p * 2
            lb = (lanes & 7) * L
            rlo_u = lb + (uidx_v[pl.ds(t * L, L)] & 15)
            rlo_i = lb + (iidx_v[pl.ds(t * L, L)] & 15)
            acc = bout_splat
            for d in range(MF):
                dsp = jnp.full((L,), d, jnp.int32)
                ug_d = plsc.load_gather(sug, [slotq, dsp, rlo_u])
                ig_d = plsc.load_gather(sig, [slotq, dsp, rlo_i])
                acc = acc + (ug_d * ig_d) * w_s[nw1 + L1 + d]

            def kstep_u(k, hs):
                xa = plsc.load_gather(
                    sum_, [slotq, jnp.full((L,), 0, jnp.int32) + 2 * k,
                           rlo_u])
                xb = plsc.load_gather(
                    sum_, [slotq, jnp.full((L,), 0, jnp.int32) + 2 * k + 1,
                           rlo_u])
                hs = tuple(hs[j] + xa * w_s[j * L0 + 2 * k]
                           for j in range(L1))
                return tuple(hs[j] + xb * w_s[j * L0 + 2 * k + 1]
                             for j in range(L1))

            def kstep_i(k, hs):
                xa = plsc.load_gather(
                    sim, [slotq, jnp.full((L,), 0, jnp.int32) + 2 * k,
                          rlo_i])
                xb = plsc.load_gather(
                    sim, [slotq, jnp.full((L,), 0, jnp.int32) + 2 * k + 1,
                          rlo_i])
                hs = tuple(hs[j] + xa * w_s[j * L0 + E + 2 * k]
                           for j in range(L1))
                return tuple(hs[j] + xb * w_s[j * L0 + E + 2 * k + 1]
                             for j in range(L1))

            hs = lax.fori_loop(0, E // 2, kstep_u, b1_splat)
            hs = lax.fori_loop(0, (L0 - E) // 2, kstep_i, hs)
            for j in range(L1):
                acc = acc + jnp.maximum(hs[j], 0.0) * w_s[nw1 + L1 + MF + j]
            out_v[pl.ds(t * L, L)] = acc

        fire(0, 0)

        def body(t, _):
            p = lax.rem(t, 2)

            @pl.when(t + 1 < nblk)
            def _():
                fire(t + 1, 1 - p)

            drain(p)
            compute(t, p)
            return 0

        lax.fori_loop(0, nblk, body, 0)
        pltpu.sync_copy(out_v, out_hbm.at[pl.ds(base, bpw)])

    return neumf


def kernel(user_indices, item_indices, ue_gmf, ie_gmf, ue_mlp, ie_mlp,
           W1, b1, Wout, bout):
    B = user_indices.shape[0]
    MF = ue_gmf.shape[1]
    E = ue_mlp.shape[1]
    L1, L0 = W1.shape
    w_all = jnp.concatenate(
        [W1.reshape(-1), b1.reshape(-1), Wout.reshape(-1), bout.reshape(-1)])
    w_all = jnp.pad(w_all, (0, NWPAD - w_all.shape[0]))
    fn = _neumf_sc(B, MF, E, L0, L1)
    out = fn(user_indices.astype(jnp.int32), item_indices.astype(jnp.int32),
             ue_gmf.T, ie_gmf.T, ue_mlp.T, ie_mlp.T, w_all)
    return out.reshape(B, 1)
